# baseline (speedup 1.0000x reference)
import functools

import jax
import jax.numpy as jnp
from jax import lax
from jax.experimental import pallas as pl
from jax.experimental.pallas import tpu as pltpu

B, N, C, D = 16, 1024, 384, 2048
K_A, K_B = 20, 4
LAMBDA_PARAM = 25.0
ALPHA = 0.25
EPS = 1e-4

_DOT = dict(preferred_element_type=jnp.float32,
            precision=jax.lax.Precision.HIGHEST)


def _vicreg_global(za, zb):
    inv_g = jnp.sum((za - zb) ** 2, keepdims=True) / (B * D)

    def half(x):
        mu = jnp.mean(x, axis=0, keepdims=True)
        xc = x - mu
        ss = jnp.sum(xc * xc, axis=0, keepdims=True)
        std = jnp.sqrt(ss / (B - 1) + EPS)
        var_l = jnp.sum(jnp.maximum(1.0 - std, 0.0), keepdims=True) / D
        gram = lax.dot_general(xc, xc, (((1,), (1,)), ((), ())), **_DOT)
        fro2 = jnp.sum(gram * gram, keepdims=True)
        diag2 = jnp.sum(ss * ss, keepdims=True)
        cov_l = (fro2 - diag2) / ((B - 1) * (B - 1) * D)
        return var_l, cov_l

    va, ca = half(za)
    vb, cb = half(zb)
    return 25.0 * inv_g + 12.5 * (va + vb) + (ca + cb)


def _kern(a_ref, b_ref, gax_ref, gay_ref, gbx_ref, gby_ref,
          za_ref, zb_ref, out_ref, s_ref):
    bi = pl.program_id(0)

    a = a_ref[0]
    bm = b_ref[0]
    af = a.astype(jnp.float32)
    bf = bm.astype(jnp.float32)
    a2 = jnp.sum(af * af, axis=1, keepdims=True)
    b2 = jnp.sum(bf * bf, axis=1, keepdims=True).T
    f = lax.dot_general(jnp.bfloat16(-2.0) * a, bm, (((1,), (1,)), ((), ())),
                        preferred_element_type=jnp.float32)
    d2f = jnp.maximum(a2 + (b2 + f), 0.0)

    rmin_f = jnp.min(d2f, axis=1, keepdims=True)
    cmin_f = jnp.min(d2f, axis=0, keepdims=True)

    gax, gay = gax_ref[0], gay_ref[0]
    gbx, gby = gbx_ref[0], gby_ref[0]
    ga2 = gax * gax + gay * gay
    gb2 = gbx * gbx + gby * gby
    g = jnp.sqrt(jnp.maximum(ga2 + gb2 - 2.0 * (gax * gbx + gay * gby), 0.0))

    gm_a = jnp.min(g, axis=1, keepdims=True)
    fsel_a = jnp.min(jnp.where(g == gm_a, d2f, jnp.inf), axis=1, keepdims=True)

    gm_b = jnp.min(g, axis=0, keepdims=True)
    fsel_b = jnp.min(jnp.where(g == gm_b, d2f, jnp.inf), axis=0, keepdims=True)

    rows = (rmin_f.T, cmin_f, gm_a.T, gm_b, fsel_a.T, fsel_b)
    for l, row in enumerate(rows):
        s_ref[pl.ds(bi + 16 * l, 1), :] = row

    @pl.when(bi == B - 1)
    def _():
        keys = s_ref[0:64, :]
        pays = jnp.concatenate([s_ref[0:32, :], s_ref[64:96, :]], axis=0)
        col = lax.broadcasted_iota(jnp.int32, (64, N), 1)
        row = lax.broadcasted_iota(jnp.int32, (64, 1), 0)
        big = jnp.int32(2**30)
        gb_rows = row >= 48

        def body(r, carry):
            k, acc = carry
            m = jnp.min(k, axis=1, keepdims=True)
            j = jnp.min(jnp.where(k == m, col, big), axis=1, keepdims=True)
            sel = col == j
            pay = jnp.sum(jnp.where(sel, pays, 0.0), axis=1, keepdims=True)
            w = jnp.where(gb_rows & (r >= K_B), 0.0, 1.0)
            return jnp.where(sel, jnp.inf, k), acc + pay * w

        _, acc = lax.fori_loop(0, K_A, body,
                               (keys, jnp.zeros((64, 1), jnp.float32)))

        c20 = (1.0 - ALPHA) * LAMBDA_PARAM / (2.0 * B * K_A * C)
        c4 = (1.0 - ALPHA) * LAMBDA_PARAM / (2.0 * B * K_B * C)
        s20 = jnp.sum(jnp.where(gb_rows, 0.0, acc), keepdims=True)
        s4 = jnp.sum(jnp.where(gb_rows, acc, 0.0), keepdims=True)
        glob = _vicreg_global(za_ref[...], zb_ref[...])
        out_ref[...] = s20 * c20 + s4 * c4 + ALPHA * glob


@functools.partial(jax.jit, static_argnames=())
def kernel(z_a, z_b, z_a_local_features, z_b_local_features, grid_a, grid_b):
    a = z_a_local_features.reshape(B, N, C).astype(jnp.bfloat16)
    bm = z_b_local_features.reshape(B, N, C).astype(jnp.bfloat16)
    ga = grid_a.reshape(B, N, 2)
    gb = grid_b.reshape(B, N, 2)
    gax = ga[..., 0:1]
    gay = ga[..., 1:2]
    gbx = gb[..., 0][:, None, :]
    gby = gb[..., 1][:, None, :]

    out = pl.pallas_call(
        _kern,
        grid=(B,),
        in_specs=[
            pl.BlockSpec((1, N, C), lambda i: (i, 0, 0)),
            pl.BlockSpec((1, N, C), lambda i: (i, 0, 0)),
            pl.BlockSpec((1, N, 1), lambda i: (i, 0, 0)),
            pl.BlockSpec((1, N, 1), lambda i: (i, 0, 0)),
            pl.BlockSpec((1, 1, N), lambda i: (i, 0, 0)),
            pl.BlockSpec((1, 1, N), lambda i: (i, 0, 0)),
            pl.BlockSpec((B, D), lambda i: (0, 0)),
            pl.BlockSpec((B, D), lambda i: (0, 0)),
        ],
        out_specs=pl.BlockSpec((1, 1), lambda i: (0, 0)),
        out_shape=jax.ShapeDtypeStruct((1, 1), jnp.float32),
        scratch_shapes=[pltpu.VMEM((96, N), jnp.float32)],
    )(a, bm, gax, gay, gbx, gby, z_a, z_b)
    return out[0, 0]


# speedup vs baseline: 1.2492x; 1.2492x over previous
import functools

import jax
import jax.numpy as jnp
from jax import lax
from jax.experimental import pallas as pl
from jax.experimental.pallas import tpu as pltpu

B, N, C, D = 16, 1024, 384, 2048
K_A, K_B = 20, 4
LAMBDA_PARAM = 25.0
ALPHA = 0.25
EPS = 1e-4

_DOT = dict(preferred_element_type=jnp.float32,
            precision=jax.lax.Precision.HIGHEST)


def _vicreg_global(za, zb):
    inv_g = jnp.sum((za - zb) ** 2, keepdims=True) / (B * D)

    def half(x):
        mu = jnp.mean(x, axis=0, keepdims=True)
        xc = x - mu
        ss = jnp.sum(xc * xc, axis=0, keepdims=True)
        std = jnp.sqrt(ss / (B - 1) + EPS)
        var_l = jnp.sum(jnp.maximum(1.0 - std, 0.0), keepdims=True) / D
        gram = lax.dot_general(xc, xc, (((1,), (1,)), ((), ())), **_DOT)
        fro2 = jnp.sum(gram * gram, keepdims=True)
        diag2 = jnp.sum(ss * ss, keepdims=True)
        cov_l = (fro2 - diag2) / ((B - 1) * (B - 1) * D)
        return var_l, cov_l

    va, ca = half(za)
    vb, cb = half(zb)
    return 25.0 * inv_g + 12.5 * (va + vb) + (ca + cb)


def _kern(a_ref, b_ref, gax_ref, gay_ref, gbx_ref, gby_ref,
          za_ref, zb_ref, out_ref, s_ref):
    bi = pl.program_id(0)

    a = a_ref[0]
    bm = b_ref[0]
    ones_r = jnp.ones((1, C), jnp.float32)
    _f32 = dict(preferred_element_type=jnp.float32)
    a2 = lax.dot_general(a * a, ones_r, (((1,), (1,)), ((), ())), **_f32)
    b2 = lax.dot_general(ones_r, bm * bm, (((1,), (1,)), ((), ())), **_f32)
    f = lax.dot_general(-2.0 * a, bm, (((1,), (1,)), ((), ())), **_f32)
    d2f = jnp.maximum(a2 + (b2 + f), 0.0)

    rmin_f = jnp.min(d2f, axis=1, keepdims=True)
    cmin_f = jnp.min(d2f, axis=0, keepdims=True)

    gax, gay = gax_ref[0], gay_ref[0]
    gbx, gby = gbx_ref[0], gby_ref[0]
    ga2 = gax * gax + gay * gay
    gb2 = gbx * gbx + gby * gby
    g = jnp.sqrt(jnp.maximum(ga2 + gb2 - 2.0 * (gax * gbx + gay * gby), 0.0))

    gm_a = jnp.min(g, axis=1, keepdims=True)
    fsel_a = jnp.min(jnp.where(g == gm_a, d2f, jnp.inf), axis=1, keepdims=True)

    gm_b = jnp.min(g, axis=0, keepdims=True)
    fsel_b = jnp.min(jnp.where(g == gm_b, d2f, jnp.inf), axis=0, keepdims=True)

    rows = (rmin_f.T, cmin_f, gm_a.T, gm_b, fsel_a.T, fsel_b)
    for l, row in enumerate(rows):
        s_ref[pl.ds(bi + 16 * l, 1), :] = row

    @pl.when(bi == B - 1)
    def _():
        keys = s_ref[0:64, :]
        pays = jnp.concatenate([s_ref[0:32, :], s_ref[64:96, :]], axis=0)
        col = lax.broadcasted_iota(jnp.int32, (64, N), 1)
        row = lax.broadcasted_iota(jnp.int32, (64, 1), 0)
        big = jnp.int32(2**30)
        gb_rows = row >= 48

        def body(r, carry):
            k, acc = carry
            m = jnp.min(k, axis=1, keepdims=True)
            j = jnp.min(jnp.where(k == m, col, big), axis=1, keepdims=True)
            sel = col == j
            pay = jnp.sum(jnp.where(sel, pays, 0.0), axis=1, keepdims=True)
            w = jnp.where(gb_rows & (r >= K_B), 0.0, 1.0)
            return jnp.where(sel, jnp.inf, k), acc + pay * w

        _, acc = lax.fori_loop(0, K_A, body,
                               (keys, jnp.zeros((64, 1), jnp.float32)))

        c20 = (1.0 - ALPHA) * LAMBDA_PARAM / (2.0 * B * K_A * C)
        c4 = (1.0 - ALPHA) * LAMBDA_PARAM / (2.0 * B * K_B * C)
        s20 = jnp.sum(jnp.where(gb_rows, 0.0, acc), keepdims=True)
        s4 = jnp.sum(jnp.where(gb_rows, acc, 0.0), keepdims=True)
        glob = _vicreg_global(za_ref[...], zb_ref[...])
        out_ref[...] = s20 * c20 + s4 * c4 + ALPHA * glob


@functools.partial(jax.jit, static_argnames=())
def kernel(z_a, z_b, z_a_local_features, z_b_local_features, grid_a, grid_b):
    a = z_a_local_features.reshape(B, N, C)
    bm = z_b_local_features.reshape(B, N, C)
    ga = grid_a.reshape(B, N, 2)
    gb = grid_b.reshape(B, N, 2)
    gax = ga[..., 0:1]
    gay = ga[..., 1:2]
    gbx = gb[..., 0][:, None, :]
    gby = gb[..., 1][:, None, :]

    out = pl.pallas_call(
        _kern,
        grid=(B,),
        in_specs=[
            pl.BlockSpec((1, N, C), lambda i: (i, 0, 0)),
            pl.BlockSpec((1, N, C), lambda i: (i, 0, 0)),
            pl.BlockSpec((1, N, 1), lambda i: (i, 0, 0)),
            pl.BlockSpec((1, N, 1), lambda i: (i, 0, 0)),
            pl.BlockSpec((1, 1, N), lambda i: (i, 0, 0)),
            pl.BlockSpec((1, 1, N), lambda i: (i, 0, 0)),
            pl.BlockSpec((B, D), lambda i: (0, 0)),
            pl.BlockSpec((B, D), lambda i: (0, 0)),
        ],
        out_specs=pl.BlockSpec((1, 1), lambda i: (0, 0)),
        out_shape=jax.ShapeDtypeStruct((1, 1), jnp.float32),
        scratch_shapes=[pltpu.VMEM((96, N), jnp.float32)],
    )(a, bm, gax, gay, gbx, gby, z_a, z_b)
    return out[0, 0]


# speedup vs baseline: 1.4229x; 1.1390x over previous
import functools

import jax
import jax.numpy as jnp
from jax import lax
from jax.experimental import pallas as pl
from jax.experimental.pallas import tpu as pltpu

B, N, C, D = 16, 1024, 384, 2048
K_A, K_B = 20, 4
LAMBDA_PARAM = 25.0
ALPHA = 0.25
EPS = 1e-4

_DOT = dict(preferred_element_type=jnp.float32,
            precision=jax.lax.Precision.HIGHEST)


def _vicreg_global(za, zb):
    inv_g = jnp.sum((za - zb) ** 2, keepdims=True) / (B * D)

    def half(x):
        mu = jnp.mean(x, axis=0, keepdims=True)
        xc = x - mu
        ss = jnp.sum(xc * xc, axis=0, keepdims=True)
        std = jnp.sqrt(ss / (B - 1) + EPS)
        var_l = jnp.sum(jnp.maximum(1.0 - std, 0.0), keepdims=True) / D
        gram = lax.dot_general(xc, xc, (((1,), (1,)), ((), ())), **_DOT)
        fro2 = jnp.sum(gram * gram, keepdims=True)
        diag2 = jnp.sum(ss * ss, keepdims=True)
        cov_l = (fro2 - diag2) / ((B - 1) * (B - 1) * D)
        return var_l, cov_l

    va, ca = half(za)
    vb, cb = half(zb)
    return 25.0 * inv_g + 12.5 * (va + vb) + (ca + cb)


def _kern(a_ref, b_ref, gax_ref, gay_ref, gbx_ref, gby_ref,
          za_ref, zb_ref, out_ref, s_ref):
    bi = pl.program_id(0)

    a = a_ref[0]
    bm = b_ref[0]
    ones_r = jnp.ones((1, C), jnp.float32)
    _f32 = dict(preferred_element_type=jnp.float32)
    a2 = lax.dot_general(a * a, ones_r, (((1,), (1,)), ((), ())), **_f32)
    b2 = lax.dot_general(ones_r, bm * bm, (((1,), (1,)), ((), ())), **_f32)
    f = lax.dot_general(-2.0 * a, bm, (((1,), (1,)), ((), ())), **_f32)
    d2f = jnp.maximum(a2 + (b2 + f), 0.0)

    rmin_f = jnp.min(d2f, axis=1, keepdims=True)
    cmin_f = jnp.min(d2f, axis=0, keepdims=True)

    gax, gay = gax_ref[0], gay_ref[0]
    gbx, gby = gbx_ref[0], gby_ref[0]
    ga2 = gax * gax + gay * gay
    gb2 = gbx * gbx + gby * gby
    g = jnp.maximum(ga2 + gb2 - 2.0 * (gax * gbx + gay * gby), 0.0)

    gm_a = jnp.min(g, axis=1, keepdims=True)
    fsel_a = jnp.min(jnp.where(g == gm_a, d2f, jnp.inf), axis=1, keepdims=True)

    gm_b = jnp.min(g, axis=0, keepdims=True)
    fsel_b = jnp.min(jnp.where(g == gm_b, d2f, jnp.inf), axis=0, keepdims=True)

    rows = (rmin_f.T, cmin_f, jnp.sqrt(gm_a).T, jnp.sqrt(gm_b),
            fsel_a.T, fsel_b)
    for l, row in enumerate(rows):
        s_ref[pl.ds(bi + 16 * l, 1), :] = row

    @pl.when(bi == B - 1)
    def _():
        keys = s_ref[0:64, :]
        pays = jnp.concatenate([s_ref[0:32, :], s_ref[64:96, :]], axis=0)
        col = lax.broadcasted_iota(jnp.int32, (64, N), 1)
        row = lax.broadcasted_iota(jnp.int32, (64, 1), 0)
        big = jnp.int32(2**30)
        gb_rows = row >= 48

        def body(r, carry):
            k, acc = carry
            m = jnp.min(k, axis=1, keepdims=True)
            j = jnp.min(jnp.where(k == m, col, big), axis=1, keepdims=True)
            sel = col == j
            pay = jnp.sum(jnp.where(sel, pays, 0.0), axis=1, keepdims=True)
            w = jnp.where(gb_rows & (r >= K_B), 0.0, 1.0)
            return jnp.where(sel, jnp.inf, k), acc + pay * w

        _, acc = lax.fori_loop(0, K_A, body,
                               (keys, jnp.zeros((64, 1), jnp.float32)))

        c20 = (1.0 - ALPHA) * LAMBDA_PARAM / (2.0 * B * K_A * C)
        c4 = (1.0 - ALPHA) * LAMBDA_PARAM / (2.0 * B * K_B * C)
        s20 = jnp.sum(jnp.where(gb_rows, 0.0, acc), keepdims=True)
        s4 = jnp.sum(jnp.where(gb_rows, acc, 0.0), keepdims=True)
        glob = _vicreg_global(za_ref[...], zb_ref[...])
        out_ref[...] = s20 * c20 + s4 * c4 + ALPHA * glob


@functools.partial(jax.jit, static_argnames=())
def kernel(z_a, z_b, z_a_local_features, z_b_local_features, grid_a, grid_b):
    a = z_a_local_features.reshape(B, N, C)
    bm = z_b_local_features.reshape(B, N, C)
    ga = grid_a.reshape(B, N, 2)
    gb = grid_b.reshape(B, N, 2)
    gax = ga[..., 0:1]
    gay = ga[..., 1:2]
    gbx = gb[..., 0][:, None, :]
    gby = gb[..., 1][:, None, :]

    out = pl.pallas_call(
        _kern,
        grid=(B,),
        in_specs=[
            pl.BlockSpec((1, N, C), lambda i: (i, 0, 0)),
            pl.BlockSpec((1, N, C), lambda i: (i, 0, 0)),
            pl.BlockSpec((1, N, 1), lambda i: (i, 0, 0)),
            pl.BlockSpec((1, N, 1), lambda i: (i, 0, 0)),
            pl.BlockSpec((1, 1, N), lambda i: (i, 0, 0)),
            pl.BlockSpec((1, 1, N), lambda i: (i, 0, 0)),
            pl.BlockSpec((B, D), lambda i: (0, 0)),
            pl.BlockSpec((B, D), lambda i: (0, 0)),
        ],
        out_specs=pl.BlockSpec((1, 1), lambda i: (0, 0)),
        out_shape=jax.ShapeDtypeStruct((1, 1), jnp.float32),
        scratch_shapes=[pltpu.VMEM((96, N), jnp.float32)],
    )(a, bm, gax, gay, gbx, gby, z_a, z_b)
    return out[0, 0]
